# Initial kernel scaffold; baseline (speedup 1.0000x reference)
import jax
import jax.numpy as jnp
from jax.experimental import pallas as pl


def kernel(input_posA, input_posB, positions, audios):
    raise NotImplementedError("write your pallas kernel here")


# speedup vs baseline: 1.8324x; 1.8324x over previous
import functools

import jax
import jax.numpy as jnp
import numpy as np
from jax.experimental import pallas as pl
from jax.experimental.pallas import tpu as pltpu

N = 1024
L = 32768
B = 32
K = 8
THR = 10.0

N1 = 256
N2 = 128


def _knn_kernel(q_ref, pt_ref, idx_ref):
    s = jnp.zeros((2 * B, N), dtype=jnp.float32)
    for c in range(3):
        qc = q_ref[:, c:c + 1]
        pc = pt_ref[c:c + 1, :]
        d = pc - qc
        s = s + d * d
    s = jnp.sqrt(s)
    iota = jax.lax.broadcasted_iota(jnp.int32, (2 * B, N), 1)
    for k in range(K):
        m = jnp.min(s, axis=1, keepdims=True)
        sel = s == m
        idxk = jnp.min(jnp.where(sel, iota, np.int32(2**30)), axis=1,
                       keepdims=True)
        idx_ref[:, k:k + 1] = idxk
        s = jnp.where(iota == idxk, jnp.inf, s)


def _topk_indices(input_posA, input_posB, positions):
    q = jnp.concatenate([input_posA, input_posB], axis=0)
    q = jnp.pad(q, ((0, 0), (0, 5)))
    pt = jnp.pad(positions.T, ((0, 5), (0, 0)))
    return pl.pallas_call(
        _knn_kernel,
        out_shape=jax.ShapeDtypeStruct((2 * B, K), jnp.int32),
    )(q, pt)


def _main_kernel(idx_ref, *refs):
    audio_refs = refs[:2 * K]
    (f1r_ref, f1i_ref, wr_ref, wi_ref, f2r_ref, f2i_ref,
     outr_ref, outi_ref) = refs[2 * K:]

    x2 = jnp.concatenate([r[0] for r in audio_refs], axis=1)

    dot = functools.partial(
        jax.lax.dot_general,
        precision=jax.lax.Precision.HIGHEST,
        preferred_element_type=jnp.float32)

    dnum2 = (((1,), (0,)), ((), ()))
    yr = dot(f1r_ref[...], x2, dnum2)
    yi = dot(f1i_ref[...], x2, dnum2)

    zr = yr * wr_ref[...] - yi * wi_ref[...]
    zi = yr * wi_ref[...] + yi * wr_ref[...]

    zr = zr.reshape(N1 * 2 * K, N2)
    zi = zi.reshape(N1 * 2 * K, N2)
    xr = dot(zr, f2r_ref[...], dnum2) - dot(zi, f2i_ref[...], dnum2)
    xi = dot(zr, f2i_ref[...], dnum2) + dot(zi, f2r_ref[...], dnum2)
    xr = xr.reshape(N1, 2 * K, N2)
    xi = xi.reshape(N1, 2 * K, N2)

    ar_all = xr[:, :K, :]
    ai_all = xi[:, :K, :]
    br = xr[:, K:, :]
    bi = xi[:, K:, :]

    acc_r = jnp.zeros((N1, N2), dtype=jnp.float32)
    acc_i = jnp.zeros((N1, N2), dtype=jnp.float32)
    for i in range(K):
        ar = ar_all[:, i:i + 1, :]
        ai = ai_all[:, i:i + 1, :]
        recip = 1.0 / (ar * ar + ai * ai)
        num_r = br * ar + bi * ai
        num_i = bi * ar - br * ai
        rr = num_r * recip
        ri = num_i * recip
        rr = jnp.where(jnp.isnan(rr), 0.0, rr)
        ri = jnp.where(jnp.isnan(ri), 0.0, ri)
        rr = jnp.clip(rr, -THR, THR)
        ri = jnp.clip(ri, -THR, THR)
        acc_r = acc_r + jnp.sum(rr, axis=1)
        acc_i = acc_i + jnp.sum(ri, axis=1)

    outr_ref[0] = acc_r * (1.0 / (K * K))
    outi_ref[0] = acc_i * (1.0 / (K * K))


def _dft_tables():
    k1 = np.arange(N1)
    ang1 = (-2.0 * np.pi / N1) * (np.outer(k1, k1) % N1)
    f1r = np.cos(ang1).astype(np.float32)
    f1i = np.sin(ang1).astype(np.float32)
    k2 = np.arange(N2)
    ang2 = (-2.0 * np.pi / N2) * (np.outer(k2, k2) % N2)
    f2r = np.cos(ang2).astype(np.float32)
    f2i = np.sin(ang2).astype(np.float32)
    angw = (-2.0 * np.pi / L) * (np.outer(np.arange(N1), np.arange(N2)) % L)
    wr = np.tile(np.cos(angw).astype(np.float32), (1, 2 * K))
    wi = np.tile(np.sin(angw).astype(np.float32), (1, 2 * K))
    return (jnp.asarray(f1r), jnp.asarray(f1i), jnp.asarray(wr),
            jnp.asarray(wi), jnp.asarray(f2r), jnp.asarray(f2i))


def kernel(input_posA, input_posB, positions, audios):
    idx = _topk_indices(input_posA, input_posB, positions)
    idx_flat = idx.reshape(-1)

    audios3 = audios.reshape(N, N1, N2)
    f1r, f1i, wr, wi, f2r, f2i = _dft_tables()

    def audio_spec(t):
        if t < K:
            def imap(b, iref, t=t):
                return (iref[b * K + t], 0, 0)
        else:
            def imap(b, iref, t=t):
                return (iref[B * K + b * K + (t - K)], 0, 0)
        return pl.BlockSpec((1, N1, N2), imap)

    in_specs = [audio_spec(t) for t in range(2 * K)]
    in_specs += [
        pl.BlockSpec((N1, N1), lambda b, iref: (0, 0)),
        pl.BlockSpec((N1, N1), lambda b, iref: (0, 0)),
        pl.BlockSpec((N1, 2 * K * N2), lambda b, iref: (0, 0)),
        pl.BlockSpec((N1, 2 * K * N2), lambda b, iref: (0, 0)),
        pl.BlockSpec((N2, N2), lambda b, iref: (0, 0)),
        pl.BlockSpec((N2, N2), lambda b, iref: (0, 0)),
    ]
    out_specs = [
        pl.BlockSpec((1, N1, N2), lambda b, iref: (b, 0, 0)),
        pl.BlockSpec((1, N1, N2), lambda b, iref: (b, 0, 0)),
    ]

    outr, outi = pl.pallas_call(
        _main_kernel,
        grid_spec=pltpu.PrefetchScalarGridSpec(
            num_scalar_prefetch=1,
            grid=(B,),
            in_specs=in_specs,
            out_specs=out_specs,
        ),
        out_shape=[
            jax.ShapeDtypeStruct((B, N1, N2), jnp.float32),
            jax.ShapeDtypeStruct((B, N1, N2), jnp.float32),
        ],
    )(idx_flat, *([audios3] * (2 * K)), f1r, f1i, wr, wi, f2r, f2i)

    warp = outr + 1j * outi
    return warp.transpose(0, 2, 1).reshape(B, L)


# speedup vs baseline: 2.9816x; 1.6271x over previous
import functools

import jax
import jax.numpy as jnp
import numpy as np
from jax.experimental import pallas as pl
from jax.experimental.pallas import tpu as pltpu

N = 1024
L = 32768
B = 32
K = 8
THR = 10.0

N1 = 128
N2 = 256
KH = 136


def _knn_kernel(q_ref, pt_ref, idx_ref):
    s = jnp.zeros((2 * B, N), dtype=jnp.float32)
    for c in range(3):
        qc = q_ref[:, c:c + 1]
        pc = pt_ref[c:c + 1, :]
        d = pc - qc
        s = s + d * d
    s = jnp.sqrt(s)
    iota = jax.lax.broadcasted_iota(jnp.int32, (2 * B, N), 1)
    for k in range(K):
        m = jnp.min(s, axis=1, keepdims=True)
        sel = s == m
        idxk = jnp.min(jnp.where(sel, iota, np.int32(2**30)), axis=1,
                       keepdims=True)
        idx_ref[:, k:k + 1] = idxk
        s = jnp.where(iota == idxk, jnp.inf, s)


def _topk_indices(input_posA, input_posB, positions):
    q = jnp.concatenate([input_posA, input_posB], axis=0)
    q = jnp.pad(q, ((0, 0), (0, 5)))
    pt = jnp.pad(positions.T, ((0, 5), (0, 0)))
    return pl.pallas_call(
        _knn_kernel,
        out_shape=jax.ShapeDtypeStruct((2 * B, K), jnp.int32),
    )(q, pt)


def _main_kernel(idx_ref, *refs):
    audio_refs = refs[:2 * K]
    (f2r_ref, f2i_ref, wr_ref, wi_ref, f1r_ref, f1i_ref,
     outr_ref, outi_ref) = refs[2 * K:]

    x2 = jnp.concatenate([r[0] for r in audio_refs], axis=1)

    dot = functools.partial(
        jax.lax.dot_general,
        precision=jax.lax.Precision.HIGHEST,
        preferred_element_type=jnp.float32)
    dnum = (((1,), (0,)), ((), ()))

    gr = dot(f2r_ref[...], x2, dnum)
    gi = dot(f2i_ref[...], x2, dnum)

    zr = gr * wr_ref[...] - gi * wi_ref[...]
    zi = gr * wi_ref[...] + gi * wr_ref[...]

    zr = zr.reshape(KH * 2 * K, N1)
    zi = zi.reshape(KH * 2 * K, N1)
    xr = dot(zr, f1r_ref[...], dnum) - dot(zi, f1i_ref[...], dnum)
    xi = dot(zr, f1i_ref[...], dnum) + dot(zi, f1r_ref[...], dnum)
    xr = xr.reshape(KH, 2 * K, N1)
    xi = xi.reshape(KH, 2 * K, N1)

    ar = xr[:, :K, :]
    ai = xi[:, :K, :]
    br = xr[:, K:, :]
    bi = xi[:, K:, :]

    recip = 1.0 / (ar * ar + ai * ai)
    acc_r = jnp.zeros((KH, K, N1), dtype=jnp.float32)
    acc_i = jnp.zeros((KH, K, N1), dtype=jnp.float32)
    for s in range(K):
        if s == 0:
            brs, bis = br, bi
        else:
            brs = jnp.concatenate([br[:, s:, :], br[:, :s, :]], axis=1)
            bis = jnp.concatenate([bi[:, s:, :], bi[:, :s, :]], axis=1)
        num_r = brs * ar + bis * ai
        num_i = bis * ar - brs * ai
        rr = num_r * recip
        ri = num_i * recip
        rr = jnp.where(jnp.isnan(rr), 0.0, rr)
        ri = jnp.where(jnp.isnan(ri), 0.0, ri)
        acc_r = acc_r + jnp.clip(rr, -THR, THR)
        acc_i = acc_i + jnp.clip(ri, -THR, THR)

    outr_ref[0] = jnp.sum(acc_r, axis=1) * (1.0 / (K * K))
    outi_ref[0] = jnp.sum(acc_i, axis=1) * (1.0 / (K * K))


def _dft_tables():
    k2 = np.arange(KH)
    n2 = np.arange(N2)
    ang = (-2.0 * np.pi / N2) * (np.outer(k2, n2) % N2)
    f2r = np.cos(ang).astype(np.float32)
    f2i = np.sin(ang).astype(np.float32)
    n1 = np.arange(N1)
    angw = (-2.0 * np.pi / L) * (np.outer(k2, n1) % L)
    wr = np.tile(np.cos(angw).astype(np.float32), (1, 2 * K))
    wi = np.tile(np.sin(angw).astype(np.float32), (1, 2 * K))
    ang1 = (-2.0 * np.pi / N1) * (np.outer(n1, n1) % N1)
    f1r = np.cos(ang1).astype(np.float32)
    f1i = np.sin(ang1).astype(np.float32)
    return (jnp.asarray(f2r), jnp.asarray(f2i), jnp.asarray(wr),
            jnp.asarray(wi), jnp.asarray(f1r), jnp.asarray(f1i))


def kernel(input_posA, input_posB, positions, audios):
    idx = _topk_indices(input_posA, input_posB, positions)
    idx_flat = idx.reshape(-1)

    audios3 = audios.reshape(N, N2, N1)
    f2r, f2i, wr, wi, f1r, f1i = _dft_tables()

    def audio_spec(t):
        if t < K:
            def imap(b, iref, t=t):
                return (iref[b * K + t], 0, 0)
        else:
            def imap(b, iref, t=t):
                return (iref[B * K + b * K + (t - K)], 0, 0)
        return pl.BlockSpec((1, N2, N1), imap)

    in_specs = [audio_spec(t) for t in range(2 * K)]
    in_specs += [
        pl.BlockSpec((KH, N2), lambda b, iref: (0, 0)),
        pl.BlockSpec((KH, N2), lambda b, iref: (0, 0)),
        pl.BlockSpec((KH, 2 * K * N1), lambda b, iref: (0, 0)),
        pl.BlockSpec((KH, 2 * K * N1), lambda b, iref: (0, 0)),
        pl.BlockSpec((N1, N1), lambda b, iref: (0, 0)),
        pl.BlockSpec((N1, N1), lambda b, iref: (0, 0)),
    ]
    out_specs = [
        pl.BlockSpec((1, KH, N1), lambda b, iref: (b, 0, 0)),
        pl.BlockSpec((1, KH, N1), lambda b, iref: (b, 0, 0)),
    ]

    outr, outi = pl.pallas_call(
        _main_kernel,
        grid_spec=pltpu.PrefetchScalarGridSpec(
            num_scalar_prefetch=1,
            grid=(B,),
            in_specs=in_specs,
            out_specs=out_specs,
        ),
        out_shape=[
            jax.ShapeDtypeStruct((B, KH, N1), jnp.float32),
            jax.ShapeDtypeStruct((B, KH, N1), jnp.float32),
        ],
    )(idx_flat, *([audios3] * (2 * K)), f2r, f2i, wr, wi, f1r, f1i)

    half = outr + 1j * outi
    part1 = half[:, :129, :]
    part2 = jnp.conj(half[:, 1:128, :])[:, ::-1, ::-1]
    full = jnp.concatenate([part1, part2], axis=1)
    return full.transpose(0, 2, 1).reshape(B, L)
